# baseline (speedup 1.0000x reference)
import functools

import jax
import jax.numpy as jnp
from jax import lax
from jax.experimental import pallas as pl
from jax.experimental.pallas import tpu as pltpu
from jax.experimental.pallas import tpu_sc as plsc

_LANES = 16
_CHUNK = 128
_RING = 4


def _make_sc_kernel(dw, e_pad, chunks_per_worker):
  mesh = plsc.VectorSubcoreMesh(core_axis_name="c", subcore_axis_name="s")
  num_cores = mesh.num_cores
  n = chunks_per_worker
  assert n % _RING == 0 and n >= _RING

  scratch = (
      [pltpu.VMEM((_CHUNK,), jnp.int32) for _ in range(_RING)]
      + [pltpu.VMEM((_CHUNK,), jnp.int32) for _ in range(_RING)]
      + [pltpu.VMEM((_CHUNK, dw), jnp.int32) for _ in range(_RING)]
      + [pltpu.VMEM((_CHUNK, dw), jnp.int32) for _ in range(_RING)]
      + [pltpu.VMEM((_CHUNK,), jnp.float32) for _ in range(_RING)]
      + [pltpu.SemaphoreType.DMA] * (3 * _RING)
  )

  @functools.partial(
      pl.kernel,
      out_type=jax.ShapeDtypeStruct((e_pad,), jnp.float32),
      mesh=mesh,
      scratch_types=scratch,
      compiler_params=pltpu.CompilerParams(use_tc_tiling_on_sc=False),
  )
  def k(xu, xb, ui, bi, out, *bufs):
    uidx = bufs[0:_RING]
    bidx = bufs[_RING:2 * _RING]
    urows = bufs[2 * _RING:3 * _RING]
    brows = bufs[3 * _RING:4 * _RING]
    outv = bufs[4 * _RING:5 * _RING]
    isem = bufs[5 * _RING:6 * _RING]
    gsem = bufs[6 * _RING:7 * _RING]
    osem = bufs[7 * _RING:8 * _RING]

    wid = lax.axis_index("s") * num_cores + lax.axis_index("c")
    tile_base = wid * (n * _CHUNK)

    def chunk_base(c):
      cc = jnp.minimum(c, n - 1)
      return tile_base + cc * _CHUNK

    def fire_idx(j, c):
      base = chunk_base(c)
      pltpu.async_copy(ui.at[pl.ds(base, _CHUNK)], uidx[j], isem[j])
      pltpu.async_copy(bi.at[pl.ds(base, _CHUNK)], bidx[j], isem[j])

    def wait_idx(j):
      pltpu.make_async_copy(ui.at[pl.ds(0, _CHUNK)], uidx[j], isem[j]).wait()
      pltpu.make_async_copy(bi.at[pl.ds(0, _CHUNK)], bidx[j], isem[j]).wait()

    def fire_gather(j):
      pltpu.async_copy(xu.at[uidx[j]], urows[j], gsem[j])
      pltpu.async_copy(xb.at[bidx[j]], brows[j], gsem[j])

    def wait_gather(j):
      pltpu.make_async_copy(xu.at[uidx[j]], urows[j], gsem[j]).wait()
      pltpu.make_async_copy(xb.at[bidx[j]], brows[j], gsem[j]).wait()

    def wait_out(j):
      pltpu.make_async_copy(
          outv[j], out.at[pl.ds(0, _CHUNK)], osem[j]).wait()

    lane_iota = lax.iota(jnp.int32, _LANES)
    shuffle_dnums = lax.GatherDimensionNumbers(
        offset_dims=(), collapsed_slice_dims=(0,), start_index_map=(0,))

    def _shuffle(v, perm):
      return lax.gather(
          v, perm[:, None], shuffle_dnums, (1,),
          indices_are_sorted=False, unique_indices=False,
          mode=lax.GatherScatterMode.PROMISE_IN_BOUNDS)

    def combine(a, b, s):
      m = (lane_iota & s) == 0
      return jnp.where(m, a, b) + _shuffle(jnp.where(m, b, a), lane_iota ^ s)

    def compute(j, c, t):
      ur, br = urows[j], brows[j]
      for grp in range(_CHUNK // _LANES):
        partials = {}
        for i in range(_LANES):
          e = grp * _LANES + i
          acc = None
          for q in range(dw // _LANES):
            ui = ur[e, pl.ds(q * _LANES, _LANES)]
            bi = br[e, pl.ds(q * _LANES, _LANES)]
            prod = (lax.bitcast_convert_type(ui << 16, jnp.float32) *
                    lax.bitcast_convert_type(bi << 16, jnp.float32) +
                    lax.bitcast_convert_type(ui, jnp.float32) *
                    lax.bitcast_convert_type(bi, jnp.float32))
            acc = prod if acc is None else acc + prod
          lvl = 0
          while lvl in partials:
            acc = combine(partials.pop(lvl), acc, 1 << lvl)
            lvl += 1
          partials[lvl] = acc
        vecs = [partials[4]]
        if grp == 0:
          @pl.when(t >= 1)
          def _():
            wait_out(j)
        outv[j][pl.ds(grp * _LANES, _LANES)] = vecs[0]
      pltpu.async_copy(outv[j], out.at[pl.ds(chunk_base(c), _CHUNK)], osem[j])

    for j in range(_RING):
      fire_idx(j, j)
    for j in range(2):
      wait_idx(j)
      fire_gather(j)

    def iter_body(t, carry):
      for j in range(_RING):
        c = _RING * t + j
        j2 = (j + 2) % _RING
        wait_gather(j)
        wait_idx(j2)
        fire_gather(j2)
        fire_idx(j, c + _RING)
        compute(j, c, t)
      return carry

    lax.fori_loop(0, n // _RING, iter_body, 0)

    wait_idx(2)
    wait_idx(3)
    wait_gather(0)
    wait_gather(1)
    for j in range(_RING):
      wait_out(j)

  return k


@jax.jit
def kernel(x_user, x_book, edge_label_index):
  d = x_user.shape[1]
  e = edge_label_index.shape[1]

  info = plsc.get_sparse_core_info()
  n_workers = info.num_cores * info.num_subcores
  devs = jax.devices()
  nd = len(devs)
  per_worker = -(-e // (nd * n_workers * _CHUNK))
  per_worker = -(-per_worker // _RING) * _RING
  e_pad = nd * n_workers * per_worker * _CHUNK

  u_idx = jnp.pad(edge_label_index[0], (0, e_pad - e))
  b_idx = jnp.pad(edge_label_index[1], (0, e_pad - e))

  def to_packed(x):
    x16 = x.astype(jnp.bfloat16)
    return lax.bitcast_convert_type(
        x16.reshape(x.shape[0], d // 2, 2), jnp.int32)

  k = _make_sc_kernel(d // 2, e_pad // nd, per_worker)
  if nd == 1:
    out = k(to_packed(x_user), to_packed(x_book), u_idx, b_idx)
  else:
    from jax.sharding import Mesh, PartitionSpec as P
    mesh = Mesh(devs, ("d",))
    out = jax.shard_map(
        k, mesh=mesh,
        in_specs=(P(), P(), P("d"), P("d")),
        out_specs=P("d"),
    )(to_packed(x_user), to_packed(x_book), u_idx, b_idx)
  return out[:e]


# speedup vs baseline: 1.3615x; 1.3615x over previous
import functools

import jax
import jax.numpy as jnp
from jax import lax
from jax.experimental import pallas as pl
from jax.experimental.pallas import tpu as pltpu
from jax.experimental.pallas import tpu_sc as plsc

_LANES = 16
_CHUNK = 128
_RING = 4


def _make_sc_kernel(dw, e_pad, chunks_per_worker):
  mesh = plsc.VectorSubcoreMesh(core_axis_name="c", subcore_axis_name="s")
  num_cores = mesh.num_cores
  n = chunks_per_worker
  assert n % _RING == 0 and n >= _RING

  scratch = (
      [pltpu.VMEM((_CHUNK,), jnp.int32) for _ in range(_RING)]
      + [pltpu.VMEM((_CHUNK,), jnp.int32) for _ in range(_RING)]
      + [pltpu.VMEM((_CHUNK, dw), jnp.int32) for _ in range(_RING)]
      + [pltpu.VMEM((_CHUNK, dw), jnp.int32) for _ in range(_RING)]
      + [pltpu.VMEM((_CHUNK,), jnp.float32) for _ in range(_RING)]
      + [pltpu.SemaphoreType.DMA] * (3 * _RING)
  )

  @functools.partial(
      pl.kernel,
      out_type=jax.ShapeDtypeStruct((e_pad,), jnp.float32),
      mesh=mesh,
      scratch_types=scratch,
      compiler_params=pltpu.CompilerParams(use_tc_tiling_on_sc=False),
  )
  def k(xu, xb, ui, bi, out, *bufs):
    uidx = bufs[0:_RING]
    bidx = bufs[_RING:2 * _RING]
    urows = bufs[2 * _RING:3 * _RING]
    brows = bufs[3 * _RING:4 * _RING]
    outv = bufs[4 * _RING:5 * _RING]
    isem = bufs[5 * _RING:6 * _RING]
    gsem = bufs[6 * _RING:7 * _RING]
    osem = bufs[7 * _RING:8 * _RING]

    wid = lax.axis_index("s") * num_cores + lax.axis_index("c")
    tile_base = wid * (n * _CHUNK)

    def chunk_base(c):
      cc = jnp.minimum(c, n - 1)
      return tile_base + cc * _CHUNK

    def fire_idx(j, c):
      base = chunk_base(c)
      pltpu.async_copy(ui.at[pl.ds(base, _CHUNK)], uidx[j], isem[j])
      pltpu.async_copy(bi.at[pl.ds(base, _CHUNK)], bidx[j], isem[j])

    def wait_idx(j):
      pltpu.make_async_copy(ui.at[pl.ds(0, _CHUNK)], uidx[j], isem[j]).wait()
      pltpu.make_async_copy(bi.at[pl.ds(0, _CHUNK)], bidx[j], isem[j]).wait()

    def fire_gather(j):
      pltpu.async_copy(xu.at[uidx[j]], urows[j], gsem[j])
      pltpu.async_copy(xb.at[bidx[j]], brows[j], gsem[j])

    def wait_gather(j):
      pltpu.make_async_copy(xu.at[uidx[j]], urows[j], gsem[j]).wait()
      pltpu.make_async_copy(xb.at[bidx[j]], brows[j], gsem[j]).wait()

    def wait_out(j):
      pltpu.make_async_copy(
          outv[j], out.at[pl.ds(0, _CHUNK)], osem[j]).wait()

    lane_iota = lax.iota(jnp.int32, _LANES)
    shuffle_dnums = lax.GatherDimensionNumbers(
        offset_dims=(), collapsed_slice_dims=(0,), start_index_map=(0,))

    def _shuffle(v, perm):
      return lax.gather(
          v, perm[:, None], shuffle_dnums, (1,),
          indices_are_sorted=False, unique_indices=False,
          mode=lax.GatherScatterMode.PROMISE_IN_BOUNDS)

    def combine(a, b, s):
      m = (lane_iota & s) == 0
      return jnp.where(m, a, b) + _shuffle(jnp.where(m, b, a), lane_iota ^ s)

    def compute(j, c, t):
      ur, br = urows[j], brows[j]
      for grp in range(_CHUNK // _LANES):
        partials = {}
        for i in range(_LANES):
          e = grp * _LANES + i
          acc = None
          for q in range(dw // _LANES):
            ui = ur[e, pl.ds(q * _LANES, _LANES)]
            bi = br[e, pl.ds(q * _LANES, _LANES)]
            prod = (lax.bitcast_convert_type(ui << 16, jnp.float32) *
                    lax.bitcast_convert_type(bi << 16, jnp.float32) +
                    lax.bitcast_convert_type(ui, jnp.float32) *
                    lax.bitcast_convert_type(bi, jnp.float32))
            acc = prod if acc is None else acc + prod
          lvl = 0
          while lvl in partials:
            acc = combine(partials.pop(lvl), acc, 1 << lvl)
            lvl += 1
          partials[lvl] = acc
        vecs = [partials[4]]
        if grp == 0:
          @pl.when(t >= 1)
          def _():
            wait_out(j)
        outv[j][pl.ds(grp * _LANES, _LANES)] = vecs[0]
      pltpu.async_copy(outv[j], out.at[pl.ds(chunk_base(c), _CHUNK)], osem[j])

    for j in range(_RING):
      fire_idx(j, j)
    for j in range(2):
      wait_idx(j)
      fire_gather(j)

    def iter_body(t, carry):
      for j in range(_RING):
        c = _RING * t + j
        j2 = (j + 2) % _RING
        wait_gather(j)
        wait_idx(j2)
        fire_gather(j2)
        fire_idx(j, c + _RING)
        compute(j, c, t)
      return carry

    lax.fori_loop(0, n // _RING, iter_body, 0)

    wait_idx(2)
    wait_idx(3)
    wait_gather(0)
    wait_gather(1)
    for j in range(_RING):
      wait_out(j)

  return k


@jax.jit
def kernel(x_user, x_book, edge_label_index):
  d = x_user.shape[1]
  e = edge_label_index.shape[1]

  info = plsc.get_sparse_core_info()
  n_workers = info.num_cores * info.num_subcores
  per_worker = -(-e // (n_workers * _CHUNK))
  per_worker = -(-per_worker // _RING) * _RING
  e_pad = n_workers * per_worker * _CHUNK

  u_idx = jnp.pad(edge_label_index[0], (0, e_pad - e))
  b_idx = jnp.pad(edge_label_index[1], (0, e_pad - e))

  def to_packed(x):
    x16 = x.astype(jnp.bfloat16)
    return lax.bitcast_convert_type(
        x16.reshape(x.shape[0], d // 2, 2), jnp.int32)

  k = _make_sc_kernel(d // 2, e_pad, per_worker)
  out = k(to_packed(x_user), to_packed(x_book), u_idx, b_idx)
  return out[:e]


# speedup vs baseline: 1.7730x; 1.3022x over previous
import functools

import jax
import jax.numpy as jnp
from jax import lax
from jax.experimental import pallas as pl
from jax.experimental.pallas import tpu as pltpu
from jax.experimental.pallas import tpu_sc as plsc

_LANES = 16
_CHUNK = 128
_RING = 4


def _make_sc_kernel(dw, e_pad, chunks_per_worker):
  mesh = plsc.VectorSubcoreMesh(core_axis_name="c", subcore_axis_name="s")
  num_cores = mesh.num_cores
  n = chunks_per_worker
  assert n % _RING == 0 and n >= _RING

  scratch = (
      [pltpu.VMEM((_CHUNK,), jnp.int32) for _ in range(_RING)]
      + [pltpu.VMEM((_CHUNK,), jnp.int32) for _ in range(_RING)]
      + [pltpu.VMEM((_CHUNK, dw), jnp.int32) for _ in range(_RING)]
      + [pltpu.VMEM((_CHUNK, dw), jnp.int32) for _ in range(_RING)]
      + [pltpu.VMEM((_CHUNK,), jnp.float32) for _ in range(_RING)]
      + [pltpu.SemaphoreType.DMA] * (3 * _RING)
  )

  @functools.partial(
      pl.kernel,
      out_type=jax.ShapeDtypeStruct((e_pad,), jnp.float32),
      mesh=mesh,
      scratch_types=scratch,
      compiler_params=pltpu.CompilerParams(use_tc_tiling_on_sc=False),
  )
  def k(xu, xb, ui, bi, out, *bufs):
    uidx = bufs[0:_RING]
    bidx = bufs[_RING:2 * _RING]
    urows = bufs[2 * _RING:3 * _RING]
    brows = bufs[3 * _RING:4 * _RING]
    outv = bufs[4 * _RING:5 * _RING]
    isem = bufs[5 * _RING:6 * _RING]
    gsem = bufs[6 * _RING:7 * _RING]
    osem = bufs[7 * _RING:8 * _RING]

    wid = lax.axis_index("s") * num_cores + lax.axis_index("c")
    tile_base = wid * (n * _CHUNK)

    def chunk_base(c):
      cc = jnp.minimum(c, n - 1)
      return tile_base + cc * _CHUNK

    def fire_idx(j, c):
      base = chunk_base(c)
      pltpu.async_copy(ui.at[pl.ds(base, _CHUNK)], uidx[j], isem[j])
      pltpu.async_copy(bi.at[pl.ds(base, _CHUNK)], bidx[j], isem[j])

    def wait_idx(j):
      pltpu.make_async_copy(ui.at[pl.ds(0, _CHUNK)], uidx[j], isem[j]).wait()
      pltpu.make_async_copy(bi.at[pl.ds(0, _CHUNK)], bidx[j], isem[j]).wait()

    def fire_gather(j):
      pltpu.async_copy(xu.at[uidx[j]], urows[j], gsem[j])
      pltpu.async_copy(xb.at[bidx[j]], brows[j], gsem[j])

    def wait_gather(j):
      pltpu.make_async_copy(xu.at[uidx[j]], urows[j], gsem[j]).wait()
      pltpu.make_async_copy(xb.at[bidx[j]], brows[j], gsem[j]).wait()

    def wait_out(j):
      pltpu.make_async_copy(
          outv[j], out.at[pl.ds(0, _CHUNK)], osem[j]).wait()

    lane_iota = lax.iota(jnp.int32, _LANES)
    shuffle_dnums = lax.GatherDimensionNumbers(
        offset_dims=(), collapsed_slice_dims=(0,), start_index_map=(0,))

    def _shuffle(v, perm):
      return lax.gather(
          v, perm[:, None], shuffle_dnums, (1,),
          indices_are_sorted=False, unique_indices=False,
          mode=lax.GatherScatterMode.PROMISE_IN_BOUNDS)

    def combine(a, b, s):
      m = (lane_iota & s) == 0
      return jnp.where(m, a, b) + _shuffle(jnp.where(m, b, a), lane_iota ^ s)

    def compute(j, c, t):
      ur, br = urows[j], brows[j]
      for grp in range(_CHUNK // _LANES):
        partials = {}
        for i in range(_LANES):
          e = grp * _LANES + i
          acc = None
          for q in range(dw // _LANES):
            ui = ur[e, pl.ds(q * _LANES, _LANES)]
            bi = br[e, pl.ds(q * _LANES, _LANES)]
            prod = (lax.bitcast_convert_type(ui << 16, jnp.float32) *
                    lax.bitcast_convert_type(bi << 16, jnp.float32) +
                    lax.bitcast_convert_type(ui, jnp.float32) *
                    lax.bitcast_convert_type(bi, jnp.float32))
            acc = prod if acc is None else acc + prod
          lvl = 0
          while lvl in partials:
            acc = combine(partials.pop(lvl), acc, 1 << lvl)
            lvl += 1
          partials[lvl] = acc
        vecs = [partials[4]]
        if grp == 0:
          @pl.when(t >= 1)
          def _():
            wait_out(j)
        outv[j][pl.ds(grp * _LANES, _LANES)] = vecs[0]
      pltpu.async_copy(outv[j], out.at[pl.ds(chunk_base(c), _CHUNK)], osem[j])

    for j in range(_RING):
      fire_idx(j, j)
    for j in range(2):
      wait_idx(j)
      fire_gather(j)

    def iter_body(t, carry):
      for j in range(_RING):
        c = _RING * t + j
        j2 = (j + 2) % _RING
        wait_gather(j)
        wait_idx(j2)
        fire_gather(j2)
        fire_idx(j, c + _RING)
        compute(j, c, t)
      return carry

    lax.fori_loop(0, n // _RING, iter_body, 0)

    wait_idx(2)
    wait_idx(3)
    wait_gather(0)
    wait_gather(1)
    for j in range(_RING):
      wait_out(j)

  return k


@jax.jit
def kernel(x_user, x_book, edge_label_index):
  d = x_user.shape[1]
  e = edge_label_index.shape[1]

  info = plsc.get_sparse_core_info()
  n_workers = info.num_cores * info.num_subcores
  per_worker = -(-e // (n_workers * _CHUNK))
  per_worker = -(-per_worker // _RING) * _RING
  e_pad = n_workers * per_worker * _CHUNK

  u_idx = jnp.pad(edge_label_index[0], (0, e_pad - e))
  b_idx = jnp.pad(edge_label_index[1], (0, e_pad - e))

  def to_packed(x):
    xu = lax.bitcast_convert_type(x, jnp.uint32) + jnp.uint32(0x8000)
    lo = xu[:, :d // 2] >> jnp.uint32(16)
    hi = xu[:, d // 2:] & jnp.uint32(0xFFFF0000)
    return lax.bitcast_convert_type(lo | hi, jnp.int32)

  k = _make_sc_kernel(d // 2, e_pad, per_worker)
  out = k(to_packed(x_user), to_packed(x_book), u_idx, b_idx)
  return out[:e]


# speedup vs baseline: 2.3745x; 1.3393x over previous
import functools

import jax
import jax.numpy as jnp
from jax import lax
from jax.experimental import pallas as pl
from jax.experimental.pallas import tpu as pltpu
from jax.experimental.pallas import tpu_sc as plsc

_LANES = 16
_CHUNK = 128
_RING = 4


def _make_pack_kernel(n_rows, d, n_workers):
  mesh = plsc.VectorSubcoreMesh(core_axis_name="c", subcore_axis_name="s")
  num_cores = mesh.num_cores
  dw = d // 2
  rows_per_tile = n_rows // n_workers
  assert n_rows % n_workers == 0
  rchunk = rows_per_tile
  while rchunk * d * 4 + rchunk * dw * 4 > 400_000:
    for f in (5, 2, 3, 7):
      if rchunk % f == 0:
        rchunk //= f
        break
    else:
      raise ValueError(rows_per_tile)
  nchunks = rows_per_tile // rchunk

  @functools.partial(
      pl.kernel,
      out_type=[
          jax.ShapeDtypeStruct((n_rows, dw), jnp.int32),
          jax.ShapeDtypeStruct((n_rows, dw), jnp.int32),
      ],
      mesh=mesh,
      scratch_types=[
          pltpu.VMEM((rchunk, d), jnp.float32),
          pltpu.VMEM((rchunk, dw), jnp.int32),
      ],
      compiler_params=pltpu.CompilerParams(use_tc_tiling_on_sc=False),
  )
  def k(xu, xb, pu, pb, inbuf, outbuf):
    wid = lax.axis_index("s") * num_cores + lax.axis_index("c")
    row0 = wid * rows_per_tile

    def pack_rows(r, carry):
      for q in range(dw // _LANES):
        a = lax.bitcast_convert_type(
            inbuf[r, pl.ds(q * _LANES, _LANES)], jnp.uint32)
        b = lax.bitcast_convert_type(
            inbuf[r, pl.ds(dw + q * _LANES, _LANES)], jnp.uint32)
        lo = (a + jnp.uint32(0x8000)) >> jnp.uint32(16)
        hi = (b + jnp.uint32(0x8000)) & jnp.uint32(0xFFFF0000)
        outbuf[r, pl.ds(q * _LANES, _LANES)] = lax.bitcast_convert_type(
            lo | hi, jnp.int32)
      return carry

    for src, dst in ((xu, pu), (xb, pb)):
      def chunk_body(c, carry, src=src, dst=dst):
        base = row0 + c * rchunk
        pltpu.sync_copy(src.at[pl.ds(base, rchunk)], inbuf)
        lax.fori_loop(0, rchunk, pack_rows, 0, unroll=4)
        pltpu.sync_copy(outbuf, dst.at[pl.ds(base, rchunk)])
        return carry

      lax.fori_loop(0, nchunks, chunk_body, 0)

  return k


def _make_gather_kernel(dw, e_total, chunks_per_worker):
  mesh = plsc.VectorSubcoreMesh(core_axis_name="c", subcore_axis_name="s")
  num_cores = mesh.num_cores
  n = chunks_per_worker
  assert n % _RING == 0 and n >= _RING

  scratch = (
      [pltpu.VMEM((_CHUNK,), jnp.int32) for _ in range(_RING)]
      + [pltpu.VMEM((_CHUNK,), jnp.int32) for _ in range(_RING)]
      + [pltpu.VMEM((_CHUNK, dw), jnp.int32) for _ in range(_RING)]
      + [pltpu.VMEM((_CHUNK, dw), jnp.int32) for _ in range(_RING)]
      + [pltpu.VMEM((_CHUNK,), jnp.float32) for _ in range(_RING)]
      + [pltpu.SemaphoreType.DMA] * (3 * _RING)
  )

  @functools.partial(
      pl.kernel,
      out_type=jax.ShapeDtypeStruct((e_total,), jnp.float32),
      mesh=mesh,
      scratch_types=scratch,
      compiler_params=pltpu.CompilerParams(use_tc_tiling_on_sc=False),
  )
  def k(xu, xb, eidx, out, *bufs):
    uidx = bufs[0:_RING]
    bidx = bufs[_RING:2 * _RING]
    urows = bufs[2 * _RING:3 * _RING]
    brows = bufs[3 * _RING:4 * _RING]
    outv = bufs[4 * _RING:5 * _RING]
    isem = bufs[5 * _RING:6 * _RING]
    gsem = bufs[6 * _RING:7 * _RING]
    osem = bufs[7 * _RING:8 * _RING]

    wid = lax.axis_index("s") * num_cores + lax.axis_index("c")
    tile_base = wid * (n * _CHUNK)

    def chunk_base(c):
      cc = jnp.minimum(c, n - 1)
      return jnp.minimum(tile_base + cc * _CHUNK, e_total - _CHUNK)

    def fire_idx(j, c):
      base = chunk_base(c)
      pltpu.async_copy(eidx.at[0, pl.ds(base, _CHUNK)], uidx[j], isem[j])
      pltpu.async_copy(eidx.at[1, pl.ds(base, _CHUNK)], bidx[j], isem[j])

    def wait_idx(j):
      pltpu.make_async_copy(
          eidx.at[0, pl.ds(0, _CHUNK)], uidx[j], isem[j]).wait()
      pltpu.make_async_copy(
          eidx.at[1, pl.ds(0, _CHUNK)], bidx[j], isem[j]).wait()

    def fire_gather(j):
      pltpu.async_copy(xu.at[uidx[j]], urows[j], gsem[j])
      pltpu.async_copy(xb.at[bidx[j]], brows[j], gsem[j])

    def wait_gather(j):
      pltpu.make_async_copy(xu.at[uidx[j]], urows[j], gsem[j]).wait()
      pltpu.make_async_copy(xb.at[bidx[j]], brows[j], gsem[j]).wait()

    def wait_out(j):
      pltpu.make_async_copy(
          outv[j], out.at[pl.ds(0, _CHUNK)], osem[j]).wait()

    lane_iota = lax.iota(jnp.int32, _LANES)
    shuffle_dnums = lax.GatherDimensionNumbers(
        offset_dims=(), collapsed_slice_dims=(0,), start_index_map=(0,))

    def _shuffle(v, perm):
      return lax.gather(
          v, perm[:, None], shuffle_dnums, (1,),
          indices_are_sorted=False, unique_indices=False,
          mode=lax.GatherScatterMode.PROMISE_IN_BOUNDS)

    def combine(a, b, s):
      m = (lane_iota & s) == 0
      return jnp.where(m, a, b) + _shuffle(jnp.where(m, b, a), lane_iota ^ s)

    def compute(j, c, t):
      ur, br = urows[j], brows[j]
      for grp in range(_CHUNK // _LANES):
        partials = {}
        for i in range(_LANES):
          e = grp * _LANES + i
          acc = None
          for q in range(dw // _LANES):
            ui = ur[e, pl.ds(q * _LANES, _LANES)]
            bi = br[e, pl.ds(q * _LANES, _LANES)]
            prod = (lax.bitcast_convert_type(ui << 16, jnp.float32) *
                    lax.bitcast_convert_type(bi << 16, jnp.float32) +
                    lax.bitcast_convert_type(ui, jnp.float32) *
                    lax.bitcast_convert_type(bi, jnp.float32))
            acc = prod if acc is None else acc + prod
          lvl = 0
          while lvl in partials:
            acc = combine(partials.pop(lvl), acc, 1 << lvl)
            lvl += 1
          partials[lvl] = acc
        if grp == 0:
          @pl.when(t >= 1)
          def _():
            wait_out(j)
        outv[j][pl.ds(grp * _LANES, _LANES)] = partials[4]
      pltpu.async_copy(outv[j], out.at[pl.ds(chunk_base(c), _CHUNK)], osem[j])

    for j in range(_RING):
      fire_idx(j, j)
    for j in range(2):
      wait_idx(j)
      fire_gather(j)

    def iter_body(t, carry):
      for j in range(_RING):
        c = _RING * t + j
        j2 = (j + 2) % _RING
        wait_gather(j)
        wait_idx(j2)
        fire_gather(j2)
        fire_idx(j, c + _RING)
        compute(j, c, t)
      return carry

    lax.fori_loop(0, n // _RING, iter_body, 0)

    wait_idx(2)
    wait_idx(3)
    wait_gather(0)
    wait_gather(1)
    for j in range(_RING):
      wait_out(j)

  return k


@jax.jit
def kernel(x_user, x_book, edge_label_index):
  d = x_user.shape[1]
  e = edge_label_index.shape[1]

  info = plsc.get_sparse_core_info()
  n_workers = info.num_cores * info.num_subcores
  per_worker = -(-e // (n_workers * _CHUNK))
  per_worker = -(-per_worker // _RING) * _RING

  pack = _make_pack_kernel(x_user.shape[0], d, n_workers)
  pu, pb = pack(x_user, x_book)

  k = _make_gather_kernel(d // 2, e, per_worker)
  return k(pu, pb, edge_label_index)


# speedup vs baseline: 2.5188x; 1.0608x over previous
import functools

import jax
import jax.numpy as jnp
from jax import lax
from jax.experimental import pallas as pl
from jax.experimental.pallas import tpu as pltpu
from jax.experimental.pallas import tpu_sc as plsc

_LANES = 16
_CHUNK = 128
_RING = 4


def _make_pack_kernel(n_rows, d, n_workers):
  mesh = plsc.VectorSubcoreMesh(core_axis_name="c", subcore_axis_name="s")
  num_cores = mesh.num_cores
  dw = d // 2
  rows_per_tile = n_rows // n_workers
  assert n_rows % n_workers == 0
  rchunk = rows_per_tile
  while rchunk * d * 4 + rchunk * dw * 4 > 400_000:
    for f in (5, 2, 3, 7):
      if rchunk % f == 0:
        rchunk //= f
        break
    else:
      raise ValueError(rows_per_tile)
  nchunks = rows_per_tile // rchunk

  @functools.partial(
      pl.kernel,
      out_type=[
          jax.ShapeDtypeStruct((n_rows, dw), jnp.int32),
          jax.ShapeDtypeStruct((n_rows, dw), jnp.int32),
      ],
      mesh=mesh,
      scratch_types=[
          pltpu.VMEM((rchunk, d), jnp.float32),
          pltpu.VMEM((rchunk, d), jnp.float32),
          pltpu.VMEM((rchunk, dw), jnp.int32),
          pltpu.VMEM((rchunk, dw), jnp.int32),
          pltpu.SemaphoreType.DMA,
          pltpu.SemaphoreType.DMA,
          pltpu.SemaphoreType.DMA,
          pltpu.SemaphoreType.DMA,
      ],
      compiler_params=pltpu.CompilerParams(use_tc_tiling_on_sc=False),
  )
  def k(xu, xb, pu, pb, in0, in1, out0, out1, is0, is1, os0, os1):
    wid = lax.axis_index("s") * num_cores + lax.axis_index("c")
    row0 = wid * rows_per_tile
    inb, outb = (in0, in1), (out0, out1)
    isem, osem = (is0, is1), (os0, os1)

    def pack_rows(inbuf, outbuf):
      def body(r, carry):
        for q in range(dw // _LANES):
          a = lax.bitcast_convert_type(
              inbuf[r, pl.ds(q * _LANES, _LANES)], jnp.uint32)
          b = lax.bitcast_convert_type(
              inbuf[r, pl.ds(dw + q * _LANES, _LANES)], jnp.uint32)
          lo = (a + jnp.uint32(0x8000)) >> jnp.uint32(16)
          hi = (b + jnp.uint32(0x8000)) & jnp.uint32(0xFFFF0000)
          outbuf[r, pl.ds(q * _LANES, _LANES)] = lax.bitcast_convert_type(
              lo | hi, jnp.int32)
        return carry
      lax.fori_loop(0, rchunk, body, 0, unroll=4)

    jobs = [(src, dst, c) for src, dst in ((xu, pu), (xb, pb))
            for c in range(nchunks)]

    def fire_in(b, src, c):
      pltpu.async_copy(
          src.at[pl.ds(row0 + c * rchunk, rchunk)], inb[b], isem[b])

    def wait_in(b, src):
      pltpu.make_async_copy(
          src.at[pl.ds(row0, rchunk)], inb[b], isem[b]).wait()

    def fire_out(b, dst, c):
      pltpu.async_copy(
          outb[b], dst.at[pl.ds(row0 + c * rchunk, rchunk)], osem[b])

    def wait_out(b, dst):
      pltpu.make_async_copy(
          outb[b], dst.at[pl.ds(row0, rchunk)], osem[b]).wait()

    fire_in(0, jobs[0][0], jobs[0][2])
    for idx, (src, dst, c) in enumerate(jobs):
      b = idx % 2
      if idx + 1 < len(jobs):
        nsrc, _, nc = jobs[idx + 1]
        fire_in(1 - b, nsrc, nc)
      wait_in(b, src)
      if idx >= 2:
        wait_out(b, jobs[idx - 2][1])
      pack_rows(inb[b], outb[b])
      fire_out(b, dst, c)
    wait_out(len(jobs) % 2, jobs[-2][1])
    wait_out((len(jobs) + 1) % 2, jobs[-1][1])

  return k


def _make_gather_kernel(dw, e_total, chunks_per_worker):
  mesh = plsc.VectorSubcoreMesh(core_axis_name="c", subcore_axis_name="s")
  num_cores = mesh.num_cores
  n = chunks_per_worker
  assert n % _RING == 0 and n >= _RING

  scratch = (
      [pltpu.VMEM((_CHUNK,), jnp.int32) for _ in range(_RING)]
      + [pltpu.VMEM((_CHUNK,), jnp.int32) for _ in range(_RING)]
      + [pltpu.VMEM((_CHUNK, dw), jnp.int32) for _ in range(_RING)]
      + [pltpu.VMEM((_CHUNK, dw), jnp.int32) for _ in range(_RING)]
      + [pltpu.VMEM((_CHUNK,), jnp.float32) for _ in range(_RING)]
      + [pltpu.SemaphoreType.DMA] * (3 * _RING)
  )

  @functools.partial(
      pl.kernel,
      out_type=jax.ShapeDtypeStruct((e_total,), jnp.float32),
      mesh=mesh,
      scratch_types=scratch,
      compiler_params=pltpu.CompilerParams(use_tc_tiling_on_sc=False),
  )
  def k(xu, xb, eidx, out, *bufs):
    uidx = bufs[0:_RING]
    bidx = bufs[_RING:2 * _RING]
    urows = bufs[2 * _RING:3 * _RING]
    brows = bufs[3 * _RING:4 * _RING]
    outv = bufs[4 * _RING:5 * _RING]
    isem = bufs[5 * _RING:6 * _RING]
    gsem = bufs[6 * _RING:7 * _RING]
    osem = bufs[7 * _RING:8 * _RING]

    wid = lax.axis_index("s") * num_cores + lax.axis_index("c")
    tile_base = wid * (n * _CHUNK)

    def chunk_base(c):
      cc = jnp.minimum(c, n - 1)
      return jnp.minimum(tile_base + cc * _CHUNK, e_total - _CHUNK)

    def fire_idx(j, c):
      base = chunk_base(c)
      pltpu.async_copy(eidx.at[0, pl.ds(base, _CHUNK)], uidx[j], isem[j])
      pltpu.async_copy(eidx.at[1, pl.ds(base, _CHUNK)], bidx[j], isem[j])

    def wait_idx(j):
      pltpu.make_async_copy(
          eidx.at[0, pl.ds(0, _CHUNK)], uidx[j], isem[j]).wait()
      pltpu.make_async_copy(
          eidx.at[1, pl.ds(0, _CHUNK)], bidx[j], isem[j]).wait()

    def fire_gather(j):
      pltpu.async_copy(xu.at[uidx[j]], urows[j], gsem[j])
      pltpu.async_copy(xb.at[bidx[j]], brows[j], gsem[j])

    def wait_gather(j):
      pltpu.make_async_copy(xu.at[uidx[j]], urows[j], gsem[j]).wait()
      pltpu.make_async_copy(xb.at[bidx[j]], brows[j], gsem[j]).wait()

    def wait_out(j):
      pltpu.make_async_copy(
          outv[j], out.at[pl.ds(0, _CHUNK)], osem[j]).wait()

    lane_iota = lax.iota(jnp.int32, _LANES)
    shuffle_dnums = lax.GatherDimensionNumbers(
        offset_dims=(), collapsed_slice_dims=(0,), start_index_map=(0,))

    def _shuffle(v, perm):
      return lax.gather(
          v, perm[:, None], shuffle_dnums, (1,),
          indices_are_sorted=False, unique_indices=False,
          mode=lax.GatherScatterMode.PROMISE_IN_BOUNDS)

    def combine(a, b, s):
      m = (lane_iota & s) == 0
      return jnp.where(m, a, b) + _shuffle(jnp.where(m, b, a), lane_iota ^ s)

    def compute(j, c, t):
      ur, br = urows[j], brows[j]
      for grp in range(_CHUNK // _LANES):
        partials = {}
        for i in range(_LANES):
          e = grp * _LANES + i
          acc = None
          for q in range(dw // _LANES):
            ui = ur[e, pl.ds(q * _LANES, _LANES)]
            bi = br[e, pl.ds(q * _LANES, _LANES)]
            prod = (lax.bitcast_convert_type(ui << 16, jnp.float32) *
                    lax.bitcast_convert_type(bi << 16, jnp.float32) +
                    lax.bitcast_convert_type(ui, jnp.float32) *
                    lax.bitcast_convert_type(bi, jnp.float32))
            acc = prod if acc is None else acc + prod
          lvl = 0
          while lvl in partials:
            acc = combine(partials.pop(lvl), acc, 1 << lvl)
            lvl += 1
          partials[lvl] = acc
        if grp == 0:
          @pl.when(t >= 1)
          def _():
            wait_out(j)
        outv[j][pl.ds(grp * _LANES, _LANES)] = partials[4]
      pltpu.async_copy(outv[j], out.at[pl.ds(chunk_base(c), _CHUNK)], osem[j])

    for j in range(_RING):
      fire_idx(j, j)
    for j in range(2):
      wait_idx(j)
      fire_gather(j)

    def iter_body(t, carry):
      for j in range(_RING):
        c = _RING * t + j
        j2 = (j + 2) % _RING
        wait_gather(j)
        wait_idx(j2)
        fire_gather(j2)
        fire_idx(j, c + _RING)
        compute(j, c, t)
      return carry

    lax.fori_loop(0, n // _RING, iter_body, 0)

    wait_idx(2)
    wait_idx(3)
    wait_gather(0)
    wait_gather(1)
    for j in range(_RING):
      wait_out(j)

  return k


@jax.jit
def kernel(x_user, x_book, edge_label_index):
  d = x_user.shape[1]
  e = edge_label_index.shape[1]

  info = plsc.get_sparse_core_info()
  n_workers = info.num_cores * info.num_subcores
  per_worker = -(-e // (n_workers * _CHUNK))
  per_worker = -(-per_worker // _RING) * _RING

  pack = _make_pack_kernel(x_user.shape[0], d, n_workers)
  pu, pb = pack(x_user, x_book)

  k = _make_gather_kernel(d // 2, e, per_worker)
  return k(pu, pb, edge_label_index)
